# baseline (speedup 1.0000x reference)
import jax
import jax.numpy as jnp
from jax import lax
from jax.experimental import pallas as pl
from jax.experimental.pallas import tpu as pltpu
from jax.experimental.pallas import tpu_sc as plsc

VOCAB = 10000
EMB = 256
B = 4096
L = 200

_INFO = plsc.get_sparse_core_info()
NC = _INFO.num_cores
NS = _INFO.num_subcores
LANES = _INFO.num_lanes
NW = NC * NS
ROWS_PER_W = B // NW
G_PER_W = ROWS_PER_W // LANES


def _proj_body(emb_ref, w_ref, out_ref):
    out_ref[:] = jnp.sum(emb_ref[:] * w_ref[:], axis=1) * (1.0 / L)


_PROJ_BLOCK = 1024

_proj_call = pl.pallas_call(
    _proj_body,
    grid=(pl.cdiv(VOCAB, _PROJ_BLOCK),),
    in_specs=[
        pl.BlockSpec((_PROJ_BLOCK, EMB), lambda i: (i, 0)),
        pl.BlockSpec((1, EMB), lambda i: (0, 0)),
    ],
    out_specs=pl.BlockSpec((_PROJ_BLOCK,), lambda i: (i,)),
    out_shape=jax.ShapeDtypeStruct((VOCAB,), jnp.float32),
)


def _sc_body(proj_hbm, resp_hbm, b_hbm, out_hbm, proj_v, resp_v, b_v, out_v):
    wid = lax.axis_index("s") * NC + lax.axis_index("c")
    pltpu.sync_copy(proj_hbm, proj_v)
    pltpu.sync_copy(resp_hbm.at[pl.ds(wid * ROWS_PER_W, ROWS_PER_W)], resp_v)
    pltpu.sync_copy(b_hbm, b_v)
    bvec = b_v[...]
    lane = lax.iota(jnp.int32, LANES)
    rowvecs = [g * LANES + lane for g in range(G_PER_W)]

    def step(l, accs):
        lvec = jnp.broadcast_to(l, (LANES,))
        new = []
        for g in range(G_PER_W):
            tok = plsc.load_gather(resp_v, [rowvecs[g], lvec])
            new.append(accs[g] + plsc.load_gather(proj_v, [tok]))
        return tuple(new)

    def body(i, accs):
        return step(2 * i + 1, step(2 * i, accs))

    accs = lax.fori_loop(
        0, L // 2, body,
        tuple(jnp.zeros((LANES,), jnp.float32) for _ in range(G_PER_W)),
    )
    for g in range(G_PER_W):
        out_v[pl.ds(g * LANES, LANES)] = accs[g] + bvec
    pltpu.sync_copy(out_v, out_hbm.at[pl.ds(wid * ROWS_PER_W, ROWS_PER_W)])


_sc_call = pl.kernel(
    _sc_body,
    out_type=jax.ShapeDtypeStruct((B,), jnp.float32),
    mesh=plsc.VectorSubcoreMesh(core_axis_name="c", subcore_axis_name="s"),
    compiler_params=pltpu.CompilerParams(needs_layout_passes=False),
    scratch_types=[
        pltpu.VMEM((VOCAB,), jnp.float32),
        pltpu.VMEM((ROWS_PER_W, L), jnp.int32),
        pltpu.VMEM((LANES,), jnp.float32),
        pltpu.VMEM((ROWS_PER_W,), jnp.float32),
    ],
)


@jax.jit
def kernel(response, emb_table, W, b):
    proj = _proj_call(emb_table, W)
    b16 = jnp.broadcast_to(b, (LANES,)).astype(jnp.float32)
    out = _sc_call(proj, response, b16)
    return out.reshape(B, 1)


# speedup vs baseline: 1.2276x; 1.2276x over previous
import jax
import jax.numpy as jnp
from jax import lax
from jax.experimental import pallas as pl
from jax.experimental.pallas import tpu as pltpu
from jax.experimental.pallas import tpu_sc as plsc

VOCAB = 10000
EMB = 256
B = 4096
L = 200

_INFO = plsc.get_sparse_core_info()
NC = _INFO.num_cores
NS = _INFO.num_subcores
LANES = _INFO.num_lanes
NW = NC * NS
ROWS_PER_W = B // NW
G_PER_W = ROWS_PER_W // LANES
IDX_PER_W = ROWS_PER_W * L

_PROJ_BLOCK = 2048


def _proj_body(emb_ref, w_ref, out_ref):
    out_ref[:] = jnp.sum(emb_ref[:] * w_ref[:], axis=1) * (1.0 / L)


_proj_call = pl.pallas_call(
    _proj_body,
    grid=(pl.cdiv(VOCAB, _PROJ_BLOCK),),
    in_specs=[
        pl.BlockSpec((_PROJ_BLOCK, EMB), lambda i: (i, 0)),
        pl.BlockSpec((1, EMB), lambda i: (0, 0)),
    ],
    out_specs=pl.BlockSpec((_PROJ_BLOCK,), lambda i: (i,)),
    out_shape=jax.ShapeDtypeStruct((VOCAB,), jnp.float32),
)


def _sc_body(proj_hbm, resp_hbm, b_hbm, out_hbm, proj_v, resp_v, b_v, out_v):
    wid = lax.axis_index("s") * NC + lax.axis_index("c")
    pltpu.sync_copy(proj_hbm, proj_v)
    pltpu.sync_copy(resp_hbm.at[wid], resp_v)
    pltpu.sync_copy(b_hbm, b_v)
    bvec = b_v[...]
    lane = lax.iota(jnp.int32, LANES)
    svecs = [(g * LANES + lane) * L for g in range(G_PER_W)]

    def body(l, accs):
        new = []
        for g in range(G_PER_W):
            tok = plsc.load_gather(resp_v, [svecs[g] + l])
            new.append(accs[g] + plsc.load_gather(proj_v, [tok]))
        return tuple(new)

    accs = lax.fori_loop(
        0, L, body, tuple(jnp.zeros((LANES,), jnp.float32) for _ in range(G_PER_W))
    )
    for g in range(G_PER_W):
        out_v[pl.ds(g * LANES, LANES)] = accs[g] + bvec
    pltpu.sync_copy(out_v, out_hbm.at[pl.ds(wid * ROWS_PER_W, ROWS_PER_W)])


_sc_call = pl.kernel(
    _sc_body,
    out_type=jax.ShapeDtypeStruct((B,), jnp.float32),
    mesh=plsc.VectorSubcoreMesh(core_axis_name="c", subcore_axis_name="s"),
    compiler_params=pltpu.CompilerParams(needs_layout_passes=False),
    scratch_types=[
        pltpu.VMEM((VOCAB,), jnp.float32),
        pltpu.VMEM((IDX_PER_W,), jnp.int32),
        pltpu.VMEM((LANES,), jnp.float32),
        pltpu.VMEM((ROWS_PER_W,), jnp.float32),
    ],
)


@jax.jit
def kernel(response, emb_table, W, b):
    proj = _proj_call(emb_table, W)
    resp = response.reshape(NW, IDX_PER_W)
    b16 = jnp.broadcast_to(b, (LANES,)).astype(jnp.float32)
    out = _sc_call(proj, resp, b16)
    return out.reshape(B, 1)
